# baseline (speedup 1.0000x reference)
import functools

import jax
import jax.numpy as jnp
from jax import lax
from jax.experimental import pallas as pl
from jax.experimental.pallas import tpu as pltpu
from jax.experimental.pallas import tpu_sc as plsc

L = 16
NC = 2
NS = 16
NW = NC * NS

CHUNK_IDX = 100
EDGES_PER_CHUNK = CHUNK_IDX // 2

PTS = 1568


@functools.lru_cache(maxsize=None)
def _build_transpose(B, N, D):
    assert N % 8 == 0 and (PTS * D) % 8 == 0
    GROUPS = PTS // L

    mesh = plsc.VectorSubcoreMesh(core_axis_name="c", subcore_axis_name="s")

    @functools.partial(
        pl.kernel,
        mesh=mesh,
        compiler_params=pltpu.CompilerParams(use_tc_tiling_on_sc=False,
                                             needs_layout_passes=False),
        out_type=jax.ShapeDtypeStruct((N, B * D), jnp.float32),
        scratch_types=[
            pltpu.VMEM((PTS, D), jnp.float32),
            pltpu.VMEM((PTS, D), jnp.float32),
            pltpu.VMEM((PTS, B * D), jnp.float32),
            pltpu.SemaphoreType.DMA,
            pltpu.SemaphoreType.DMA,
        ],
    )
    def transpose(pred_hbm, table_hbm, st0, st1, out_buf, sem0, sem1):
        wid = lax.axis_index("s") * NC + lax.axis_index("c")
        p0 = jnp.minimum(wid * PTS, N - PTS)
        stages = (st0, st1)
        sems = (sem0, sem1)

        def issue(b):
            pltpu.async_copy(pred_hbm.at[b, pl.ds(p0, PTS), :],
                             stages[b % 2], sems[b % 2])

        def wait(b):
            pltpu.make_async_copy(pred_hbm.at[0, pl.ds(0, PTS), :],
                                  stages[b % 2], sems[b % 2]).wait()

        iota = lax.iota(jnp.int32, L)
        issue(0)
        for b in range(B):
            wait(b)
            if b + 1 < B:
                issue(b + 1)
            stage = stages[b % 2]

            def grp(gi, carry, b=b, stage=stage):
                p_rel = gi * L + iota
                for d in range(D):
                    col = jnp.full((L,), 3 * b + d, jnp.int32)
                    dcol = jnp.full((L,), d, jnp.int32)
                    vals = plsc.load_gather(stage, [p_rel, dcol])
                    plsc.store_scatter(out_buf, [p_rel, col], vals)
                return carry

            lax.fori_loop(0, GROUPS, grp, 0)

        pltpu.sync_copy(out_buf, table_hbm.at[pl.ds(p0, PTS), :])

    return transpose


@functools.lru_cache(maxsize=None)
def _build(n_points, bd, nchunks):
    assert bd % L == 0
    assert nchunks % 2 == 0

    mesh = plsc.VectorSubcoreMesh(core_axis_name="c", subcore_axis_name="s")

    @functools.partial(
        pl.kernel,
        mesh=mesh,
        compiler_params=pltpu.CompilerParams(use_tc_tiling_on_sc=False),
        out_type=jax.ShapeDtypeStruct((NW * L,), jnp.float32),
        scratch_types=[
            pltpu.VMEM((nchunks, CHUNK_IDX), jnp.int32),
            pltpu.VMEM((CHUNK_IDX, bd), jnp.float32),
            pltpu.VMEM((CHUNK_IDX, bd), jnp.float32),
            pltpu.VMEM((L,), jnp.float32),
            pltpu.SemaphoreType.DMA,
            pltpu.SemaphoreType.DMA,
        ],
    )
    def edge_mse(table_hbm, idx_hbm, out_hbm, idx_v, rows0, rows1, acc_v,
                 sem0, sem1):
        wid = lax.axis_index("s") * NC + lax.axis_index("c")
        pltpu.sync_copy(idx_hbm.at[wid], idx_v)

        rows = (rows0, rows1)
        sems = (sem0, sem1)

        def issue(c, b):
            pltpu.async_copy(table_hbm.at[idx_v.at[c]], rows[b], sems[b])

        def wait(b):
            pltpu.make_async_copy(table_hbm.at[idx_v.at[0]], rows[b],
                                  sems[b]).wait()

        def chunk_sum(rows_ref, acc):
            def edge(j, acc):
                s = 2 * j
                for k in range(bd // L):
                    a = rows_ref[s, pl.ds(L * k, L)]
                    b = rows_ref[s + 1, pl.ds(L * k, L)]
                    d = a - b
                    acc = acc + d * d
                return acc
            return lax.fori_loop(0, EDGES_PER_CHUNK, edge, acc, unroll=2)

        issue(0, 0)
        issue(1, 1)

        def outer(g, acc):
            for b in range(2):
                wait(b)
                acc = chunk_sum(rows[b], acc)
                issue(2 * g + 2 + b, b)
            return acc

        acc = jnp.zeros((L,), jnp.float32)
        acc = lax.fori_loop(0, nchunks // 2 - 1, outer, acc)
        for b in range(2):
            wait(b)
            acc = chunk_sum(rows[b], acc)

        acc_v[...] = acc
        pltpu.sync_copy(acc_v, out_hbm.at[pl.ds(wid * L, L)])

    return edge_mse


def kernel(pred, edges):
    B, N, D = pred.shape
    E = edges.shape[0]
    assert (2 * E) % (NW * CHUNK_IDX) == 0
    nchunks = (2 * E) // (NW * CHUNK_IDX)
    table = _build_transpose(B, N, D)(pred)
    idx = edges.reshape(NW, nchunks, CHUNK_IDX)
    partials = _build(N, B * D, nchunks)(table, idx)
    return jnp.sum(partials) / jnp.float32(B * E)


# speedup vs baseline: 1.0005x; 1.0005x over previous
import functools

import jax
import jax.numpy as jnp
from jax import lax
from jax.experimental import pallas as pl
from jax.experimental.pallas import tpu as pltpu
from jax.experimental.pallas import tpu_sc as plsc

L = 16
NC = 2
NS = 16
NW = NC * NS

CHUNK_IDX = 100
EDGES_PER_CHUNK = CHUNK_IDX // 2

PTS = 1568


@functools.lru_cache(maxsize=None)
def _build_transpose(B, N, D):
    assert N % 8 == 0 and (PTS * D) % 8 == 0
    GROUPS = PTS // L

    mesh = plsc.VectorSubcoreMesh(core_axis_name="c", subcore_axis_name="s")

    @functools.partial(
        pl.kernel,
        mesh=mesh,
        compiler_params=pltpu.CompilerParams(use_tc_tiling_on_sc=False,
                                             needs_layout_passes=False),
        out_type=jax.ShapeDtypeStruct((N, B * D), jnp.float32),
        scratch_types=[
            pltpu.VMEM((PTS, D), jnp.float32),
            pltpu.VMEM((PTS, D), jnp.float32),
            pltpu.VMEM((PTS, B * D), jnp.float32),
            pltpu.SemaphoreType.DMA,
            pltpu.SemaphoreType.DMA,
        ],
    )
    def transpose(pred_hbm, table_hbm, st0, st1, out_buf, sem0, sem1):
        wid = lax.axis_index("s") * NC + lax.axis_index("c")
        p0 = jnp.minimum(wid * PTS, N - PTS)
        stages = (st0, st1)
        sems = (sem0, sem1)

        def issue(b):
            pltpu.async_copy(pred_hbm.at[b, pl.ds(p0, PTS), :],
                             stages[b % 2], sems[b % 2])

        def wait(b):
            pltpu.make_async_copy(pred_hbm.at[0, pl.ds(0, PTS), :],
                                  stages[b % 2], sems[b % 2]).wait()

        iota = lax.iota(jnp.int32, L)
        issue(0)
        for b in range(B):
            wait(b)
            if b + 1 < B:
                issue(b + 1)
            stage = stages[b % 2]

            def grp(gi, carry, b=b, stage=stage):
                p_rel = gi * L + iota
                for d in range(D):
                    col = jnp.full((L,), 3 * b + d, jnp.int32)
                    dcol = jnp.full((L,), d, jnp.int32)
                    vals = plsc.load_gather(stage, [p_rel, dcol])
                    plsc.store_scatter(out_buf, [p_rel, col], vals)
                return carry

            lax.fori_loop(0, GROUPS, grp, 0, unroll=7)

        pltpu.sync_copy(out_buf, table_hbm.at[pl.ds(p0, PTS), :])

    return transpose


@functools.lru_cache(maxsize=None)
def _build(n_points, bd, nchunks):
    assert bd % L == 0
    assert nchunks % 2 == 0

    mesh = plsc.VectorSubcoreMesh(core_axis_name="c", subcore_axis_name="s")

    @functools.partial(
        pl.kernel,
        mesh=mesh,
        compiler_params=pltpu.CompilerParams(use_tc_tiling_on_sc=False),
        out_type=jax.ShapeDtypeStruct((NW * L,), jnp.float32),
        scratch_types=[
            pltpu.VMEM((nchunks, CHUNK_IDX), jnp.int32),
            pltpu.VMEM((CHUNK_IDX, bd), jnp.float32),
            pltpu.VMEM((CHUNK_IDX, bd), jnp.float32),
            pltpu.VMEM((L,), jnp.float32),
            pltpu.SemaphoreType.DMA,
            pltpu.SemaphoreType.DMA,
        ],
    )
    def edge_mse(table_hbm, idx_hbm, out_hbm, idx_v, rows0, rows1, acc_v,
                 sem0, sem1):
        wid = lax.axis_index("s") * NC + lax.axis_index("c")
        pltpu.sync_copy(idx_hbm.at[wid], idx_v)

        rows = (rows0, rows1)
        sems = (sem0, sem1)

        def issue(c, b):
            pltpu.async_copy(table_hbm.at[idx_v.at[c]], rows[b], sems[b])

        def wait(b):
            pltpu.make_async_copy(table_hbm.at[idx_v.at[0]], rows[b],
                                  sems[b]).wait()

        def chunk_sum(rows_ref, acc):
            def edge(j, acc):
                s = 2 * j
                for k in range(bd // L):
                    a = rows_ref[s, pl.ds(L * k, L)]
                    b = rows_ref[s + 1, pl.ds(L * k, L)]
                    d = a - b
                    acc = acc + d * d
                return acc
            return lax.fori_loop(0, EDGES_PER_CHUNK, edge, acc, unroll=2)

        issue(0, 0)
        issue(1, 1)

        def outer(g, acc):
            for b in range(2):
                wait(b)
                acc = chunk_sum(rows[b], acc)
                issue(2 * g + 2 + b, b)
            return acc

        acc = jnp.zeros((L,), jnp.float32)
        acc = lax.fori_loop(0, nchunks // 2 - 1, outer, acc)
        for b in range(2):
            wait(b)
            acc = chunk_sum(rows[b], acc)

        acc_v[...] = acc
        pltpu.sync_copy(acc_v, out_hbm.at[pl.ds(wid * L, L)])

    return edge_mse


def kernel(pred, edges):
    B, N, D = pred.shape
    E = edges.shape[0]
    assert (2 * E) % (NW * CHUNK_IDX) == 0
    nchunks = (2 * E) // (NW * CHUNK_IDX)
    table = _build_transpose(B, N, D)(pred)
    idx = edges.reshape(NW, nchunks, CHUNK_IDX)
    partials = _build(N, B * D, nchunks)(table, idx)
    return jnp.sum(partials) / jnp.float32(B * E)


# speedup vs baseline: 18.1642x; 18.1558x over previous
import functools

import jax
import jax.numpy as jnp
from jax import lax
from jax.experimental import pallas as pl
from jax.experimental.pallas import tpu as pltpu
from jax.experimental.pallas import tpu_sc as plsc

L = 16
NC = 2
NS = 16
NW = NC * NS

CHUNK_IDX = 100
EDGES_PER_CHUNK = CHUNK_IDX // 2

PTS = 800
NBLK = 2


@functools.lru_cache(maxsize=None)
def _build_rowify(F, N):
    assert F % L == 0 and PTS % L == 0 and (PTS * F) % 8 == 0
    GROUPS = PTS // L
    NTOT = NW * NBLK

    mesh = plsc.VectorSubcoreMesh(core_axis_name="c", subcore_axis_name="s")

    @functools.partial(
        pl.kernel,
        mesh=mesh,
        compiler_params=pltpu.CompilerParams(use_tc_tiling_on_sc=False,
                                             needs_layout_passes=False),
        out_type=jax.ShapeDtypeStruct((N, F), jnp.float32),
        scratch_types=[
            pltpu.VMEM((F, PTS), jnp.float32),
            pltpu.VMEM((F, PTS), jnp.float32),
            pltpu.VMEM((PTS, F), jnp.float32),
            pltpu.SemaphoreType.DMA,
            pltpu.SemaphoreType.DMA,
        ],
    )
    def rowify(planes_hbm, table_hbm, st0, st1, out_buf, sem0, sem1):
        wid = lax.axis_index("s") * NC + lax.axis_index("c")
        stages = (st0, st1)
        sems = (sem0, sem1)

        def blk_start(i):
            return jnp.minimum((wid * NBLK + i) * PTS, N - PTS)

        def issue(i):
            pltpu.async_copy(planes_hbm.at[:, pl.ds(blk_start(i), PTS)],
                             stages[i % 2], sems[i % 2])

        def wait(i):
            pltpu.make_async_copy(planes_hbm.at[:, pl.ds(0, PTS)],
                                  stages[i % 2], sems[i % 2]).wait()

        iota = lax.iota(jnp.int32, L)
        issue(0)
        for i in range(NBLK):
            wait(i)
            if i + 1 < NBLK:
                issue(i + 1)
            stage = stages[i % 2]

            def grp(g, carry, stage=stage):
                p_rel = g * L + iota
                for f in range(F):
                    vals = stage[f, pl.ds(g * L, L)]
                    fcol = jnp.full((L,), f, jnp.int32)
                    plsc.store_scatter(out_buf, [p_rel, fcol], vals)
                return carry

            lax.fori_loop(0, GROUPS, grp, 0)
            pltpu.sync_copy(out_buf,
                            table_hbm.at[pl.ds(blk_start(i), PTS), :])

    return rowify


@functools.lru_cache(maxsize=None)
def _build(n_points, bd, nblocks):
    assert bd % L == 0
    BASE = nblocks // NW
    EXTRA = nblocks - BASE * NW
    assert BASE % 2 == 1 and EXTRA < NW

    mesh = plsc.VectorSubcoreMesh(core_axis_name="c", subcore_axis_name="s")

    @functools.partial(
        pl.kernel,
        mesh=mesh,
        compiler_params=pltpu.CompilerParams(use_tc_tiling_on_sc=False),
        out_type=jax.ShapeDtypeStruct((NW * L,), jnp.float32),
        scratch_types=[
            pltpu.VMEM((BASE + 1, 2, 128), jnp.int32),
            pltpu.VMEM((128, bd), jnp.float32),
            pltpu.VMEM((128, bd), jnp.float32),
            pltpu.VMEM((128, bd), jnp.float32),
            pltpu.VMEM((128, bd), jnp.float32),
            pltpu.VMEM((L,), jnp.float32),
            pltpu.SemaphoreType.DMA,
            pltpu.SemaphoreType.DMA,
            pltpu.SemaphoreType.DMA,
            pltpu.SemaphoreType.DMA,
        ],
    )
    def edge_mse(table_hbm, eblk_hbm, out_hbm, est, rs0, rs1, rt0, rt1,
                 acc_v, ss0, ss1, st0, st1):
        wid = lax.axis_index("s") * NC + lax.axis_index("c")
        start = wid * BASE
        pltpu.sync_copy(eblk_hbm.at[pl.ds(start, BASE)],
                        est.at[pl.ds(0, BASE)])
        xsrc = jnp.minimum(BASE * NW + wid, nblocks - 1)
        pltpu.sync_copy(eblk_hbm.at[pl.ds(xsrc, 1)], est.at[pl.ds(BASE, 1)])
        tsel = jnp.where(wid < EXTRA, 1, 0)

        rs = (rs0, rs1)
        rt = (rt0, rt1)
        ss = (ss0, ss1)
        st = (st0, st1)

        def issue(i, b, t_idx=1):
            pltpu.async_copy(table_hbm.at[est.at[i, 0]], rs[b], ss[b])
            pltpu.async_copy(table_hbm.at[est.at[i, t_idx]], rt[b], st[b])

        def wait(b):
            pltpu.make_async_copy(table_hbm.at[est.at[0, 0]], rs[b],
                                  ss[b]).wait()
            pltpu.make_async_copy(table_hbm.at[est.at[0, 0]], rt[b],
                                  st[b]).wait()

        def blk_sum(b, acc):
            def edge(j, acc):
                for k in range(bd // L):
                    a = rs[b][j, pl.ds(L * k, L)]
                    t = rt[b][j, pl.ds(L * k, L)]
                    d = a - t
                    acc = acc + d * d
                return acc
            return lax.fori_loop(0, 128, edge, acc, unroll=2)

        issue(0, 0)
        issue(1, 1)

        def outer(g, acc):
            for b in range(2):
                wait(b)
                acc = blk_sum(b, acc)
                issue(2 * g + 2 + b, b)
            return acc

        acc = jnp.zeros((L,), jnp.float32)
        acc = lax.fori_loop(0, (BASE - 3) // 2, outer, acc)
        wait(0)
        acc = blk_sum(0, acc)
        issue(BASE - 1, 0)
        wait(1)
        acc = blk_sum(1, acc)
        issue(BASE, 1, tsel)
        wait(0)
        acc = blk_sum(0, acc)
        wait(1)
        acc = blk_sum(1, acc)

        acc_v[...] = acc
        pltpu.sync_copy(acc_v, out_hbm.at[pl.ds(wid * L, L)])

    return edge_mse


def kernel(pred, edges):
    B, N, D = pred.shape
    E = edges.shape[0]
    assert E % 128 == 0
    planes = jnp.transpose(pred, (2, 0, 1)).reshape(D * B, N)
    table = _build_rowify(D * B, N)(planes)
    eblk = jnp.transpose(edges.reshape(E // 128, 128, 2), (0, 2, 1))
    partials = _build(N, B * D, E // 128)(table, eblk)
    return jnp.sum(partials) / jnp.float32(B * E)
